# baseline (speedup 1.0000x reference)
import functools

import jax
import jax.numpy as jnp
from jax import lax
from jax.experimental import pallas as pl
from jax.experimental.pallas import tpu as pltpu
from jax.experimental.pallas import tpu_sc as plsc

G = 8
K = 2
C = 96
B = 16
HW = 56
P = HW * HW
HIDDEN = 64
LB_COEF = 0.01
NC = 2
NS = 16

RPC = HW
RPB = C * RPC
NCHUNK = 12
CCH = C // NCHUNK
RCH = CCH * RPC


def _pool_one_body(gref, pooled_ref):
    b = pl.program_id(0)
    x = gref[...].reshape(C, HW, HW)
    s1 = jnp.sum(x, axis=1)
    pooled_ref[pl.ds(b, 1), :] = (jnp.sum(s1, axis=-1) * (1.0 / P))[None, :]


def _pool_one(group2d, interpret=False):
    return pl.pallas_call(
        _pool_one_body,
        grid=(B,),
        in_specs=[pl.BlockSpec((RPB, HW), lambda b: (b, 0))],
        out_specs=pl.BlockSpec((B, C), lambda b: (0, 0)),
        out_shape=jax.ShapeDtypeStruct((B, C), jnp.float32),
        interpret=interpret,
    )(group2d)


def _mlp_body(p0, p1, p2, p3, p4, p5, p6, p7, w1, b1, w2, b2,
              logits_ref, logits_t_ref, probs_ref, loss_ref):
    prefs = (p0, p1, p2, p3, p4, p5, p6, p7)
    cols = []
    for g in range(G):
        pg = prefs[g][...]
        h = jnp.maximum(
            jnp.dot(pg, w1[g], preferred_element_type=jnp.float32)
            + b1[g][None, :], 0.0)
        lgt = jnp.dot(h, w2[g], preferred_element_type=jnp.float32) \
            + b2[g][None, :]
        cols.append(lgt)
    logits = jnp.concatenate(cols, axis=1)
    logits_ref[...] = logits
    logits_t_ref[...] = logits.T
    m = jnp.max(logits, axis=1, keepdims=True)
    e = jnp.exp(logits - m)
    probs = e / jnp.sum(e, axis=1, keepdims=True)
    probs_ref[...] = probs
    imp = jnp.mean(probs, axis=0)
    loss_ref[...] = jnp.full((1, 1), LB_COEF * G) * jnp.sum(imp * imp)


def _mlp(pooled, w1, b1, w2, b2, interpret=False):
    f32 = jnp.float32
    return pl.pallas_call(
        _mlp_body,
        out_shape=[
            jax.ShapeDtypeStruct((B, G), f32),
            jax.ShapeDtypeStruct((G, B), f32),
            jax.ShapeDtypeStruct((B, G), f32),
            jax.ShapeDtypeStruct((1, 1), f32),
        ],
        interpret=interpret,
    )(*pooled, w1, b1, w2, b2)


def _pool_mlp(groups2d, w1, b1, w2, b2, interpret=False):
    pooled = [_pool_one(g2, interpret=interpret) for g2 in groups2d]
    return _mlp(pooled, w1, b1, w2, b2, interpret=interpret)


def _route_body(lg_t_hbm, mask_t_hbm, i1_hbm, i2_hbm,
                lg_v, mk_v, i1_v, i2_v):
    wid = lax.axis_index("s") * NC + lax.axis_index("c")

    pltpu.sync_copy(lg_t_hbm, lg_v)

    neg = jnp.full((16,), -3.0e38, jnp.float32)
    m1 = neg
    i1 = jnp.zeros((16,), jnp.int32)
    for g in range(G):
        v = lg_v[g]
        better = v > m1
        m1 = jnp.where(better, v, m1)
        i1 = jnp.where(better, g, i1)
    m2 = neg
    i2 = jnp.zeros((16,), jnp.int32)
    for g in range(G):
        v = lg_v[g]
        ok = (v > m2) & (i1 != g)
        m2 = jnp.where(ok, v, m2)
        i2 = jnp.where(ok, g, i2)

    for g in range(G):
        sel = (i1 == g) | (i2 == g)
        mk_v[g] = jnp.where(sel, 1.0, 0.0).astype(jnp.float32)
    i1_v[...] = i1
    i2_v[...] = i2

    @pl.when(wid == 0)
    def _():
        pltpu.sync_copy(mk_v, mask_t_hbm)
        pltpu.sync_copy(i1_v, i1_hbm)
        pltpu.sync_copy(i2_v, i2_hbm)


def _route(logits_t):
    f32 = jnp.float32
    run = pl.kernel(
        _route_body,
        out_type=[
            jax.ShapeDtypeStruct((G, B), f32),
            jax.ShapeDtypeStruct((B,), jnp.int32),
            jax.ShapeDtypeStruct((B,), jnp.int32),
        ],
        mesh=plsc.VectorSubcoreMesh(core_axis_name="c", subcore_axis_name="s"),
        compiler_params=pltpu.CompilerParams(needs_layout_passes=False),
        scratch_types=[
            pltpu.VMEM((G, 16), f32),
            pltpu.VMEM((G, 16), f32),
            pltpu.VMEM((16,), jnp.int32),
            pltpu.VMEM((16,), jnp.int32),
        ],
    )
    return run(logits_t)


def _sel_of(w, i1r, i2r):
    b, k = divmod(w, K)
    return i1r[b] if k == 0 else i2r[b]


GSPL = 4
CG = C // GSPL
RG = RPB // GSPL


def _gather_body(i1r, i2r, g0, g1, g2, g3, g4, g5, g6, g7, out_ref):
    grefs = (g0, g1, g2, g3, g4, g5, g6, g7)
    w = pl.program_id(0)
    b = w // K
    idx = jnp.where(w % K == 0, i1r[b], i2r[b])
    for g in range(G):
        @pl.when(idx == g)
        def _(g=g):
            out_ref[...] = grefs[g][...].reshape(1, CG, HW, HW)


def _gather_in_spec(g):
    def im(w, c, i1r, i2r):
        best = jnp.int32(0)
        cur = jnp.bool_(False)
        for w2 in range(B * K):
            b2 = w2 // K
            sel = _sel_of(w2, i1r, i2r) == g
            active = (w2 <= w) & sel
            best = jnp.where(active, jnp.int32(b2), best)
            cur = jnp.where(w2 == w, sel, cur)
        return (best * GSPL + jnp.where(cur, c, GSPL - 1), 0)
    return pl.BlockSpec((RG, HW), im)


def _gather(idx1, idx2, groups2d, interpret=False):
    f32 = jnp.float32
    return pl.pallas_call(
        _gather_body,
        grid_spec=pltpu.PrefetchScalarGridSpec(
            num_scalar_prefetch=2,
            grid=(B * K, GSPL),
            in_specs=[_gather_in_spec(g) for g in range(G)],
            out_specs=pl.BlockSpec(
                (1, CG, HW, HW),
                lambda w, c, i1r, i2r: (w // K, (w % K) * GSPL + c, 0, 0)),
        ),
        out_shape=jax.ShapeDtypeStruct((B, K * C, HW, HW), f32),
        interpret=interpret,
    )(idx1, idx2, *groups2d)


def kernel(groups_0, groups_1, groups_2, groups_3, groups_4, groups_5,
           groups_6, groups_7, W1, b1, W2, b2):
    gs = (groups_0, groups_1, groups_2, groups_3, groups_4, groups_5,
          groups_6, groups_7)
    gs2 = tuple(g.reshape(B * C * HW, HW) for g in gs)
    logits, logits_t, soft_probs, loss11 = _pool_mlp(gs2, W1, b1, W2, b2)
    mask_t, idx1, idx2 = _route(logits_t)
    out = _gather(idx1, idx2, gs2)
    hard_mask = mask_t.T
    load_loss = loss11[0, 0]
    return (out, logits, hard_mask, soft_probs, load_loss)


# speedup vs baseline: 1.4532x; 1.4532x over previous
import functools

import jax
import jax.numpy as jnp
from jax import lax
from jax.experimental import pallas as pl
from jax.experimental.pallas import tpu as pltpu
from jax.experimental.pallas import tpu_sc as plsc

G = 8
K = 2
C = 96
B = 16
HW = 56
P = HW * HW
HIDDEN = 64
LB_COEF = 0.01
NC = 2
NS = 16

RPC = HW
RPB = C * RPC
NCHUNK = 12
CCH = C // NCHUNK
RCH = CCH * RPC


def _pool_one_body(gref, pooled_ref):
    b = pl.program_id(0)
    x = gref[...].reshape(C, HW, HW)
    s1 = jnp.sum(x, axis=1)
    pooled_ref[pl.ds(b, 1), :] = (jnp.sum(s1, axis=-1) * (1.0 / P))[None, :]


def _pool_one(group2d, interpret=False):
    return pl.pallas_call(
        _pool_one_body,
        grid=(B,),
        in_specs=[pl.BlockSpec((RPB, HW), lambda b: (b, 0))],
        out_specs=pl.BlockSpec((B, C), lambda b: (0, 0)),
        out_shape=jax.ShapeDtypeStruct((B, C), jnp.float32),
        interpret=interpret,
    )(group2d)


def _mlp_body(p0, p1, p2, p3, p4, p5, p6, p7, w1, b1, w2, b2,
              logits_ref, logits_t_ref, probs_ref, loss_ref):
    prefs = (p0, p1, p2, p3, p4, p5, p6, p7)
    cols = []
    for g in range(G):
        pg = prefs[g][...]
        h = jnp.maximum(
            jnp.dot(pg, w1[g], preferred_element_type=jnp.float32)
            + b1[g][None, :], 0.0)
        lgt = jnp.dot(h, w2[g], preferred_element_type=jnp.float32) \
            + b2[g][None, :]
        cols.append(lgt)
    logits = jnp.concatenate(cols, axis=1)
    logits_ref[...] = logits
    logits_t_ref[...] = logits.T
    m = jnp.max(logits, axis=1, keepdims=True)
    e = jnp.exp(logits - m)
    probs = e / jnp.sum(e, axis=1, keepdims=True)
    probs_ref[...] = probs
    imp = jnp.mean(probs, axis=0)
    loss_ref[...] = jnp.full((1, 1), LB_COEF * G) * jnp.sum(imp * imp)


def _mlp(pooled, w1, b1, w2, b2, interpret=False):
    f32 = jnp.float32
    return pl.pallas_call(
        _mlp_body,
        out_shape=[
            jax.ShapeDtypeStruct((B, G), f32),
            jax.ShapeDtypeStruct((G, B), f32),
            jax.ShapeDtypeStruct((B, G), f32),
            jax.ShapeDtypeStruct((1, 1), f32),
        ],
        interpret=interpret,
    )(*pooled, w1, b1, w2, b2)


def _pool_mlp(groups2d, w1, b1, w2, b2, interpret=False):
    pooled = [_pool_one(g2, interpret=interpret) for g2 in groups2d]
    return _mlp(pooled, w1, b1, w2, b2, interpret=interpret)


def _route_body(lg_t_hbm, mask_t_hbm, i1_hbm, i2_hbm,
                lg_v, mk_v, i1_v, i2_v):
    wid = lax.axis_index("s") * NC + lax.axis_index("c")

    pltpu.sync_copy(lg_t_hbm, lg_v)

    neg = jnp.full((16,), -3.0e38, jnp.float32)
    m1 = neg
    i1 = jnp.zeros((16,), jnp.int32)
    for g in range(G):
        v = lg_v[g]
        better = v > m1
        m1 = jnp.where(better, v, m1)
        i1 = jnp.where(better, g, i1)
    m2 = neg
    i2 = jnp.zeros((16,), jnp.int32)
    for g in range(G):
        v = lg_v[g]
        ok = (v > m2) & (i1 != g)
        m2 = jnp.where(ok, v, m2)
        i2 = jnp.where(ok, g, i2)

    for g in range(G):
        sel = (i1 == g) | (i2 == g)
        mk_v[g] = jnp.where(sel, 1.0, 0.0).astype(jnp.float32)
    i1_v[...] = i1
    i2_v[...] = i2

    @pl.when(wid == 0)
    def _():
        pltpu.sync_copy(mk_v, mask_t_hbm)
        pltpu.sync_copy(i1_v, i1_hbm)
        pltpu.sync_copy(i2_v, i2_hbm)


def _route(logits_t):
    f32 = jnp.float32
    run = pl.kernel(
        _route_body,
        out_type=[
            jax.ShapeDtypeStruct((G, B), f32),
            jax.ShapeDtypeStruct((B,), jnp.int32),
            jax.ShapeDtypeStruct((B,), jnp.int32),
        ],
        mesh=plsc.VectorSubcoreMesh(core_axis_name="c", subcore_axis_name="s"),
        compiler_params=pltpu.CompilerParams(needs_layout_passes=False),
        scratch_types=[
            pltpu.VMEM((G, 16), f32),
            pltpu.VMEM((G, 16), f32),
            pltpu.VMEM((16,), jnp.int32),
            pltpu.VMEM((16,), jnp.int32),
        ],
    )
    return run(logits_t)


def _sel_of(w, i1r, i2r):
    b, k = divmod(w, K)
    return i1r[b] if k == 0 else i2r[b]


GSPL = 1
CG = C // GSPL
RG = RPB // GSPL


def _gather_body(i1r, i2r, g0, g1, g2, g3, g4, g5, g6, g7, out_ref):
    grefs = (g0, g1, g2, g3, g4, g5, g6, g7)
    w = pl.program_id(0)
    b = w // K
    idx = jnp.where(w % K == 0, i1r[b], i2r[b])
    for g in range(G):
        @pl.when(idx == g)
        def _(g=g):
            out_ref[...] = grefs[g][...].reshape(1, CG, HW, HW)


def _gather_in_spec(g):
    def im(w, c, i1r, i2r):
        best = jnp.int32(0)
        cur = jnp.bool_(False)
        for w2 in range(B * K):
            b2 = w2 // K
            sel = _sel_of(w2, i1r, i2r) == g
            active = (w2 <= w) & sel
            best = jnp.where(active, jnp.int32(b2), best)
            cur = jnp.where(w2 == w, sel, cur)
        return (best * GSPL + jnp.where(cur, c, GSPL - 1), 0)
    return pl.BlockSpec((RG, HW), im)


def _gather(idx1, idx2, groups2d, interpret=False):
    f32 = jnp.float32
    return pl.pallas_call(
        _gather_body,
        grid_spec=pltpu.PrefetchScalarGridSpec(
            num_scalar_prefetch=2,
            grid=(B * K, GSPL),
            in_specs=[_gather_in_spec(g) for g in range(G)],
            out_specs=pl.BlockSpec(
                (1, CG, HW, HW),
                lambda w, c, i1r, i2r: (w // K, (w % K) * GSPL + c, 0, 0)),
        ),
        out_shape=jax.ShapeDtypeStruct((B, K * C, HW, HW), f32),
        interpret=interpret,
    )(idx1, idx2, *groups2d)


def kernel(groups_0, groups_1, groups_2, groups_3, groups_4, groups_5,
           groups_6, groups_7, W1, b1, W2, b2):
    gs = (groups_0, groups_1, groups_2, groups_3, groups_4, groups_5,
          groups_6, groups_7)
    gs2 = tuple(g.reshape(B * C * HW, HW) for g in gs)
    logits, logits_t, soft_probs, loss11 = _pool_mlp(gs2, W1, b1, W2, b2)
    mask_t, idx1, idx2 = _route(logits_t)
    out = _gather(idx1, idx2, gs2)
    hard_mask = mask_t.T
    load_loss = loss11[0, 0]
    return (out, logits, hard_mask, soft_probs, load_loss)
